# baseline (speedup 1.0000x reference)
import functools

import jax
import jax.numpy as jnp
from jax import lax
from jax.experimental import pallas as pl
from jax.experimental.pallas import tpu as pltpu
from jax.experimental.pallas import tpu_sc as plsc

NUM_CLASSES = 1000
NUM_EXAMPLES = 4096
NUM_FEATURES = 26
NUM_WORKERS = 32
EB = NUM_EXAMPLES // NUM_WORKERS
NBUF = 5
CW = NUM_CLASSES // NBUF
LANES = 16

_mesh = plsc.VectorSubcoreMesh(core_axis_name="c", subcore_axis_name="s")


@functools.partial(
    pl.kernel,
    out_type=jax.ShapeDtypeStruct(
        (NUM_FEATURES, NUM_CLASSES, NUM_EXAMPLES), jnp.int32),
    mesh=_mesh,
    compiler_params=pltpu.CompilerParams(needs_layout_passes=False),
    scratch_types=[
        pltpu.VMEM((2, EB), jnp.int32),
        pltpu.VMEM((CW, EB), jnp.int32),
        pltpu.VMEM((CW, EB), jnp.int32),
        pltpu.VMEM((CW, EB), jnp.int32),
        pltpu.VMEM((CW, EB), jnp.int32),
        pltpu.VMEM((CW, EB), jnp.int32),
        pltpu.SemaphoreType.DMA,
        pltpu.SemaphoreType.DMA,
        pltpu.SemaphoreType.DMA,
        pltpu.SemaphoreType.DMA,
        pltpu.SemaphoreType.DMA,
    ],
)
def _onehot_sc(idx_hbm, out_hbm, idxrow, b0, b1, b2, b3, b4,
               s0, s1, s2, s3, s4):
    wid = lax.axis_index("s") * 2 + lax.axis_index("c")
    e0 = wid * EB

    zeros16 = jnp.zeros((LANES,), jnp.int32)
    ones16 = jnp.ones((LANES,), jnp.int32)
    lane = lax.iota(jnp.int32, LANES)

    bufs = (b0, b1, b2, b3, b4)
    sems = (s0, s1, s2, s3, s4)
    groups = EB // LANES

    def memset(i, carry):
        r = i // groups
        c = (i % groups) * LANES
        for buf in bufs:
            buf[r, pl.ds(c, LANES)] = zeros16
        return carry

    lax.fori_loop(0, CW * groups, memset, 0, unroll=8)

    def load_row(f, parity):
        pltpu.sync_copy(idx_hbm.at[f, pl.ds(e0, EB)],
                        idxrow.at[parity])

    def scatter(parity, h, buf, val):
        c0 = h * CW
        for g in range(groups):
            idxv = idxrow[parity, pl.ds(g * LANES, LANES)]
            crel = idxv - c0
            mask = (crel >= 0) & (crel < CW)
            erel = lane + g * LANES
            plsc.store_scatter(buf, [crel, erel], val, mask=mask)

    def out_copy(f, h, buf, sem):
        return pltpu.make_async_copy(
            buf, out_hbm.at[f, pl.ds(h * CW, CW), pl.ds(e0, EB)], sem)

    load_row(0, 0)
    for b in range(NBUF):
        scatter(0, b, bufs[b], ones16)
        out_copy(0, b, bufs[b], sems[b]).start()

    def outer(f, carry):
        p = lax.rem(f, 2)
        q = 1 - p
        load_row(f, p)
        for b in range(NBUF):
            out_copy(f - 1, b, bufs[b], sems[b]).wait()
            scatter(q, b, bufs[b], zeros16)
            scatter(p, b, bufs[b], ones16)
            out_copy(f, b, bufs[b], sems[b]).start()
        return carry

    lax.fori_loop(1, NUM_FEATURES, outer, 0)

    for b in range(NBUF):
        out_copy(NUM_FEATURES - 1, b, bufs[b], sems[b]).wait()


def kernel(inputs):
    out_t = _onehot_sc(inputs.T)
    return jnp.transpose(out_t, (2, 0, 1))


# speedup vs baseline: 1.0276x; 1.0276x over previous
import functools

import jax
import jax.numpy as jnp
from jax import lax
from jax.experimental import pallas as pl
from jax.experimental.pallas import tpu as pltpu
from jax.experimental.pallas import tpu_sc as plsc

NUM_CLASSES = 1000
NUM_EXAMPLES = 4096
NUM_FEATURES = 26
NUM_WORKERS = 32
EB = NUM_EXAMPLES // NUM_WORKERS
NBUF = 5
CW = NUM_CLASSES // NBUF
LANES = 16

_mesh = plsc.VectorSubcoreMesh(core_axis_name="c", subcore_axis_name="s")


@functools.partial(
    pl.kernel,
    out_type=jax.ShapeDtypeStruct(
        (NUM_FEATURES, NUM_CLASSES, NUM_EXAMPLES), jnp.int32),
    mesh=_mesh,
    compiler_params=pltpu.CompilerParams(needs_layout_passes=False),
    scratch_types=[
        pltpu.VMEM((3, EB), jnp.int32),
        pltpu.VMEM((CW, EB), jnp.int32),
        pltpu.VMEM((CW, EB), jnp.int32),
        pltpu.VMEM((CW, EB), jnp.int32),
        pltpu.VMEM((CW, EB), jnp.int32),
        pltpu.VMEM((CW, EB), jnp.int32),
        pltpu.SemaphoreType.DMA,
        pltpu.SemaphoreType.DMA,
        pltpu.SemaphoreType.DMA,
        pltpu.SemaphoreType.DMA,
        pltpu.SemaphoreType.DMA,
        pltpu.SemaphoreType.DMA,
    ],
)
def _onehot_sc(idx_hbm, out_hbm, idxrow, b0, b1, b2, b3, b4,
               s0, s1, s2, s3, s4, si):
    wid = lax.axis_index("s") * 2 + lax.axis_index("c")
    e0 = wid * EB

    zeros16 = jnp.zeros((LANES,), jnp.int32)
    ones16 = jnp.ones((LANES,), jnp.int32)
    lane = lax.iota(jnp.int32, LANES)

    bufs = (b0, b1, b2, b3, b4)
    sems = (s0, s1, s2, s3, s4)
    groups = EB // LANES

    def row_copy(f, slot):
        return pltpu.make_async_copy(
            idx_hbm.at[f, pl.ds(e0, EB)], idxrow.at[slot], si)

    def scatter(slot, h, buf, val):
        c0 = h * CW
        for g in range(groups):
            idxv = idxrow[slot, pl.ds(g * LANES, LANES)]
            crel = idxv - c0
            mask = (crel >= 0) & (crel < CW)
            erel = lane + g * LANES
            plsc.store_scatter(buf, [crel, erel], val, mask=mask)

    def out_copy(f, h, buf, sem):
        return pltpu.make_async_copy(
            buf, out_hbm.at[f, pl.ds(h * CW, CW), pl.ds(e0, EB)], sem)

    row_copy(0, 0).start()

    def memset(buf):
        def step(i, carry):
            r = i // groups
            c = (i % groups) * LANES
            buf[r, pl.ds(c, LANES)] = zeros16
            return carry
        lax.fori_loop(0, CW * groups, step, 0, unroll=8)

    for b in range(NBUF):
        memset(bufs[b])
        if b == 0:
            row_copy(0, 0).wait()
        scatter(0, b, bufs[b], ones16)
        out_copy(0, b, bufs[b], sems[b]).start()
    row_copy(1, 1).start()

    def outer(f, carry):
        slot = lax.rem(f, 3)
        slot_prev = lax.rem(f - 1, 3)

        @pl.when(f < NUM_FEATURES - 1)
        def _():
            row_copy(f + 1, lax.rem(f + 1, 3)).start()

        row_copy(f, slot).wait()
        for b in range(NBUF):
            out_copy(f - 1, b, bufs[b], sems[b]).wait()
            scatter(slot_prev, b, bufs[b], zeros16)
            scatter(slot, b, bufs[b], ones16)
            out_copy(f, b, bufs[b], sems[b]).start()
        return carry

    lax.fori_loop(1, NUM_FEATURES, outer, 0)

    for b in range(NBUF):
        out_copy(NUM_FEATURES - 1, b, bufs[b], sems[b]).wait()


def kernel(inputs):
    out_t = _onehot_sc(inputs.T)
    return jnp.transpose(out_t, (2, 0, 1))
